# baseline (speedup 1.0000x reference)
import functools

import jax
import jax.numpy as jnp
from jax import lax
from jax.experimental import pallas as pl
from jax.experimental.pallas import tpu as pltpu
from jax.experimental.pallas import tpu_sc as plsc

_E_TILE = 1024
_CHUNK = 128
_NC, _NS = 2, 16


def _argmin_body(p_ref, e_ref, idx_ref, cnt_ref, *, n_events):
    Tp = p_ref.shape[0]
    p_col = p_ref[...]
    p_iota = jax.lax.broadcasted_iota(jnp.int32, (Tp, 1), 0).astype(jnp.float32)
    rows_per_tile = _E_TILE // 128

    cnt_ref[...] = jnp.zeros((Tp, 1), jnp.float32)

    def step(t, _):
        e_row = e_ref[:, pl.ds(t * _E_TILE, _E_TILE)]
        dist = jnp.abs(p_col - e_row)
        min_d = jnp.min(dist, axis=0, keepdims=True)
        masked = jnp.where(dist == min_d, p_iota, jnp.float32(Tp))
        min_idx = jnp.min(masked, axis=0, keepdims=True)
        oh_t = (p_iota == min_idx).astype(jnp.float32)
        cnt_ref[...] += jnp.sum(oh_t, axis=1, keepdims=True)
        idx_ref[pl.ds(t * rows_per_tile, rows_per_tile), :] = (
            min_idx.astype(jnp.int32).reshape(rows_per_tile, 128)
        )
        return 0

    jax.lax.fori_loop(0, n_events // _E_TILE, step, 0)


def _sc_scatter_body(idx_hbm, ev_hbm, sum_hbm,
                     idx_v, stage_v, zero_v, acc_sh):
    Tp = acc_sh.shape[1]
    cid = lax.axis_index("c")
    sid = lax.axis_index("s")
    b_loc = sid // 8
    q = sid % 8
    b = cid * 2 + b_loc
    slab = acc_sh.at[b_loc]

    zero16 = jnp.zeros((16,), jnp.float32)

    def zrow(i, _):
        for r in range(4):
            for k in range(8):
                zero_v[i * 4 + r, pl.ds(k * 16, 16)] = zero16
        return 0

    lax.fori_loop(0, zero_v.shape[0] // 4, zrow, 0)

    pltpu.sync_copy(idx_hbm.at[b, pl.ds(q * 4, 4)], idx_v)

    def do_d_pass(dp):
        c0 = dp * 128
        pltpu.sync_copy(zero_v, slab.at[pl.ds(q * 256, 256)])
        plsc.subcore_barrier()

        def chunk(j, _):
            e0 = q * 512 + j * _CHUNK
            pltpu.sync_copy(
                ev_hbm.at[b, pl.ds(e0, _CHUNK), pl.ds(c0, 128)], stage_v
            )
            pltpu.sync_copy(stage_v, slab.at[idx_v.at[j]], add=True)
            return 0

        lax.fori_loop(0, 4, chunk, 0)
        plsc.subcore_barrier()
        pltpu.sync_copy(
            slab.at[pl.ds(q * 256, 256)],
            sum_hbm.at[b, pl.ds(q * 256, 256), pl.ds(c0, 128)],
        )
        plsc.subcore_barrier()

    do_d_pass(0)
    do_d_pass(1)


def _divide_body(sum_ref, cnt_ref, out_ref):
    out_ref[...] = sum_ref[...] / jnp.maximum(cnt_ref[...], 1.0)


def _tc_argmin(p, e, Tp, Te):
    Bh = p.shape[0]
    n_rows = Te // _CHUNK
    return pl.pallas_call(
        functools.partial(_argmin_body, n_events=Te),
        grid=(Bh,),
        in_specs=[
            pl.BlockSpec((None, Tp, 1), lambda b: (b, 0, 0)),
            pl.BlockSpec((None, 1, Te), lambda b: (b, 0, 0)),
        ],
        out_specs=[
            pl.BlockSpec((None, n_rows, _CHUNK), lambda b: (b, 0, 0)),
            pl.BlockSpec((None, Tp, 1), lambda b: (b, 0, 0)),
        ],
        out_shape=[
            jax.ShapeDtypeStruct((Bh, n_rows, _CHUNK), jnp.int32),
            jax.ShapeDtypeStruct((Bh, Tp, 1), jnp.float32),
        ],
    )(p.reshape(Bh, Tp, 1), e.reshape(Bh, 1, Te))


def _sc_scatter(idx, ev, Tp, D):
    Bh = ev.shape[0]
    mesh = plsc.VectorSubcoreMesh(core_axis_name="c", subcore_axis_name="s")
    return pl.kernel(
        _sc_scatter_body,
        out_type=jax.ShapeDtypeStruct((Bh, Tp, D), jnp.float32),
        mesh=mesh,
        scratch_types=[
            pltpu.VMEM((4, _CHUNK), jnp.int32),
            pltpu.VMEM((_CHUNK, 128), jnp.float32),
            pltpu.VMEM((256, 128), jnp.float32),
            pltpu.VMEM_SHARED((2, Tp, 128), jnp.float32),
        ],
    )(idx, ev)


def _tc_divide(out_sum, counts, Tp, D):
    Bh = out_sum.shape[0]
    return pl.pallas_call(
        _divide_body,
        grid=(Bh,),
        in_specs=[
            pl.BlockSpec((None, Tp, D), lambda b: (b, 0, 0)),
            pl.BlockSpec((None, Tp, 1), lambda b: (b, 0, 0)),
        ],
        out_specs=pl.BlockSpec((None, Tp, D), lambda b: (b, 0, 0)),
        out_shape=jax.ShapeDtypeStruct((Bh, Tp, D), jnp.float32),
    )(out_sum, counts)


def kernel(price_timestamps, event_timestamps, event_values):
    B, Tp = price_timestamps.shape
    Te = event_timestamps.shape[1]
    D = event_values.shape[2]
    Bh = B // 2

    idx0, cnt0 = _tc_argmin(price_timestamps[:Bh], event_timestamps[:Bh], Tp, Te)
    sum0 = _sc_scatter(idx0, event_values[:Bh], Tp, D)
    idx1, cnt1 = _tc_argmin(price_timestamps[Bh:], event_timestamps[Bh:], Tp, Te)
    sum1 = _sc_scatter(idx1, event_values[Bh:], Tp, D)
    out0 = _tc_divide(sum0, cnt0, Tp, D)
    out1 = _tc_divide(sum1, cnt1, Tp, D)

    out = jnp.concatenate([out0, out1], axis=0)
    counts = jnp.concatenate([cnt0.reshape(Bh, Tp), cnt1.reshape(Bh, Tp)], axis=0)
    return out, counts > 0


# speedup vs baseline: 1.3946x; 1.3946x over previous
import functools

import jax
import jax.numpy as jnp
from jax.experimental import pallas as pl

_E_TILE = 2048


def _align_body(p_ref, e_ref, v_ref, out_ref, cnt_ref, *, n_events):
    Tp = p_ref.shape[0]
    D = v_ref.shape[1]
    p_col = p_ref[...]
    p_iota = jax.lax.broadcasted_iota(jnp.int32, (Tp, 1), 0).astype(jnp.float32)

    out_ref[...] = jnp.zeros((Tp, D), jnp.float32)
    cnt_ref[...] = jnp.zeros((Tp, 1), jnp.float32)

    n_tiles = n_events // _E_TILE

    def step(t, _):
        e_row = e_ref[:, pl.ds(t * _E_TILE, _E_TILE)]
        dist = jnp.abs(p_col - e_row)
        min_d = jnp.min(dist, axis=0, keepdims=True)
        masked = jnp.where(dist == min_d, p_iota, jnp.float32(Tp))
        min_idx = jnp.min(masked, axis=0, keepdims=True)
        oh_t = (p_iota == min_idx).astype(jnp.float32)
        vals = v_ref[pl.ds(t * _E_TILE, _E_TILE), :]
        out_ref[...] += jnp.dot(oh_t, vals, preferred_element_type=jnp.float32)
        cnt_ref[...] += jnp.sum(oh_t, axis=1, keepdims=True)
        return 0

    jax.lax.fori_loop(0, n_tiles, step, 0)
    out_ref[...] = out_ref[...] / jnp.maximum(cnt_ref[...], 1.0)


def kernel(price_timestamps, event_timestamps, event_values):
    B, Tp = price_timestamps.shape
    Te = event_timestamps.shape[1]
    D = event_values.shape[2]

    out, counts = pl.pallas_call(
        functools.partial(_align_body, n_events=Te),
        grid=(B,),
        in_specs=[
            pl.BlockSpec((None, Tp, 1), lambda b: (b, 0, 0)),
            pl.BlockSpec((None, 1, Te), lambda b: (b, 0, 0)),
            pl.BlockSpec((None, Te, D), lambda b: (b, 0, 0)),
        ],
        out_specs=[
            pl.BlockSpec((None, Tp, D), lambda b: (b, 0, 0)),
            pl.BlockSpec((None, Tp, 1), lambda b: (b, 0, 0)),
        ],
        out_shape=[
            jax.ShapeDtypeStruct((B, Tp, D), jnp.float32),
            jax.ShapeDtypeStruct((B, Tp, 1), jnp.float32),
        ],
    )(
        price_timestamps.reshape(B, Tp, 1),
        event_timestamps.reshape(B, 1, Te),
        event_values,
    )
    return out, counts.reshape(B, Tp) > 0
